# Initial kernel scaffold; baseline (speedup 1.0000x reference)
import jax
import jax.numpy as jnp
from jax.experimental import pallas as pl


def kernel(x, embed):
    raise NotImplementedError("write your pallas kernel here")


# speedup vs baseline: 1.0417x; 1.0417x over previous
import functools

import jax
import jax.numpy as jnp
from jax import lax
from jax.experimental import pallas as pl
from jax.experimental.pallas import tpu as pltpu
from jax.experimental.pallas import tpu_sc as plsc

DIM = 256
K = 8192
NT = 2304
KB = 512


def _argmin_body(xb_ref, eb_ref, x2_ref, e2_ref, idx_ref, best_ref, bidx_ref):
    k = pl.program_id(0)
    mm = lax.dot_general(
        xb_ref[...], eb_ref[...], (((1,), (1,)), ((), ())),
        preferred_element_type=jnp.float32)
    s = (x2_ref[...] - 2.0 * mm) + e2_ref[...]
    mn = jnp.min(s, axis=1, keepdims=True)
    iota = lax.broadcasted_iota(jnp.int32, s.shape, 1)
    cand = jnp.where(s == mn, iota, K)
    am = jnp.min(cand, axis=1, keepdims=True) + k * KB

    @pl.when(k == 0)
    def _init():
        best_ref[...] = mn
        bidx_ref[...] = am

    @pl.when(k > 0)
    def _update():
        better = mn < best_ref[...]
        best_ref[...] = jnp.where(better, mn, best_ref[...])
        bidx_ref[...] = jnp.where(better, am, bidx_ref[...])

    @pl.when(k == pl.num_programs(0) - 1)
    def _emit():
        idx_ref[...] = bidx_ref[...]


def _argmin_indices(xb, eb, x2, e2):
    return pl.pallas_call(
        _argmin_body,
        grid=(K // KB,),
        in_specs=[
            pl.BlockSpec((NT, DIM), lambda k: (0, 0)),
            pl.BlockSpec((KB, DIM), lambda k: (k, 0)),
            pl.BlockSpec((NT, 1), lambda k: (0, 0)),
            pl.BlockSpec((1, KB), lambda k: (0, k)),
        ],
        out_specs=pl.BlockSpec((NT, 1), lambda k: (0, 0)),
        out_shape=jax.ShapeDtypeStruct((NT, 1), jnp.int32),
        scratch_shapes=[
            pltpu.VMEM((NT, 1), jnp.float32),
            pltpu.VMEM((NT, 1), jnp.int32),
        ],
    )(xb, eb, x2, e2)


def _sc_gather(table, idx):
    info = plsc.get_sparse_core_info()
    nc, ns = info.num_cores, info.num_subcores
    bpw = NT // (nc * ns)
    mesh = plsc.VectorSubcoreMesh(core_axis_name="c", subcore_axis_name="s")

    @functools.partial(
        pl.kernel, mesh=mesh,
        out_type=jax.ShapeDtypeStruct((NT, DIM), jnp.float32),
        scratch_types=[
            pltpu.VMEM((bpw,), jnp.int32),
            pltpu.VMEM((bpw, DIM), jnp.float32),
            pltpu.SemaphoreType.DMA,
        ],
    )
    def gather_kernel(table_hbm, idx_hbm, out_hbm, idx_v, rows_v, sem):
        wid = lax.axis_index("s") * nc + lax.axis_index("c")
        base = wid * bpw
        pltpu.sync_copy(idx_hbm.at[pl.ds(base, bpw)], idx_v)
        pltpu.async_copy(table_hbm.at[idx_v], rows_v, sem).wait()
        pltpu.sync_copy(rows_v, out_hbm.at[pl.ds(base, bpw)])

    return gather_kernel(table, idx)


def kernel(x, embed):
    xf = x.reshape(NT, DIM)
    ef = embed.reshape(K, DIM)
    x2 = jnp.sum(x * x, axis=-1).reshape(NT, 1)
    e2 = jnp.sum(embed * embed, axis=-1).reshape(1, K)
    idx = _argmin_indices(
        xf.astype(jnp.bfloat16), ef.astype(jnp.bfloat16), x2, e2)
    idx_flat = idx.reshape(NT)
    quant = _sc_gather(ef, idx_flat)
    return quant.reshape(x.shape), idx_flat.reshape(x.shape[:-1])


# speedup vs baseline: 1.4711x; 1.4122x over previous
import functools

import jax
import jax.numpy as jnp
from jax import lax
from jax.experimental import pallas as pl
from jax.experimental.pallas import tpu as pltpu
from jax.experimental.pallas import tpu_sc as plsc

DIM = 256
K = 8192
NT = 2304
KB = 1024
TG = 8
LC = 128


def _argmin_body(x_ref, e_ref, x2_ref, e2_ref, idx_ref,
                 xb_s, mm_s, best_ref, bidx_ref):
    k = pl.program_id(0)

    @pl.when(k == 0)
    def _init():
        xb_s[...] = x_ref[...].astype(jnp.bfloat16)
        best_ref[...] = jnp.full((NT, LC), jnp.inf, jnp.float32)
        bidx_ref[...] = jnp.zeros((NT, LC), jnp.int32)

    mm_s[...] = lax.dot_general(
        xb_s[...], e_ref[...].astype(jnp.bfloat16),
        (((1,), (1,)), ((), ())),
        preferred_element_type=jnp.float32)

    nchunk = KB // LC
    x2 = x2_ref[...]
    iota = lax.broadcasted_iota(jnp.int32, (NT, LC), 1)
    b = best_ref[...]
    bi = bidx_ref[...]
    for c in range(nchunk):
        mmc = mm_s[:, pl.ds(c * LC, LC)]
        s = (x2 - 2.0 * mmc) + e2_ref[:, pl.ds(c * LC, LC)]
        m = s < b
        b = jnp.where(m, s, b)
        bi = jnp.where(m, iota + (k * KB + c * LC), bi)
    best_ref[...] = b
    bidx_ref[...] = bi

    @pl.when(k == pl.num_programs(0) - 1)
    def _emit():
        b = best_ref[...]
        mn = jnp.min(b, axis=1, keepdims=True)
        cand = jnp.where(b == mn, bidx_ref[...], K)
        idx_ref[...] = jnp.min(cand, axis=1, keepdims=True)


def _argmin_indices(xf, ef, x2, e2):
    return pl.pallas_call(
        _argmin_body,
        grid=(K // KB,),
        in_specs=[
            pl.BlockSpec((NT, DIM), lambda k: (0, 0)),
            pl.BlockSpec((KB, DIM), lambda k: (k, 0)),
            pl.BlockSpec((NT, 1), lambda k: (0, 0)),
            pl.BlockSpec((1, KB), lambda k: (0, k)),
        ],
        out_specs=pl.BlockSpec((NT, 1), lambda k: (0, 0)),
        out_shape=jax.ShapeDtypeStruct((NT, 1), jnp.int32),
        scratch_shapes=[
            pltpu.VMEM((NT, DIM), jnp.bfloat16),
            pltpu.VMEM((NT, KB), jnp.float32),
            pltpu.VMEM((NT, LC), jnp.float32),
            pltpu.VMEM((NT, LC), jnp.int32),
        ],
    )(xf, ef, x2, e2)


def _sc_gather(table, idx):
    info = plsc.get_sparse_core_info()
    nc, ns = info.num_cores, info.num_subcores
    bpw = NT // (nc * ns)
    mesh = plsc.VectorSubcoreMesh(core_axis_name="c", subcore_axis_name="s")

    @functools.partial(
        pl.kernel, mesh=mesh,
        out_type=jax.ShapeDtypeStruct((NT, DIM), jnp.float32),
        scratch_types=[
            pltpu.VMEM((bpw,), jnp.int32),
            pltpu.VMEM((bpw, DIM), jnp.float32),
            pltpu.SemaphoreType.DMA,
        ],
    )
    def gather_kernel(table_hbm, idx_hbm, out_hbm, idx_v, rows_v, sem):
        wid = lax.axis_index("s") * nc + lax.axis_index("c")
        base = wid * bpw
        pltpu.sync_copy(idx_hbm.at[pl.ds(base, bpw)], idx_v)
        pltpu.async_copy(table_hbm.at[idx_v], rows_v, sem).wait()
        pltpu.sync_copy(rows_v, out_hbm.at[pl.ds(base, bpw)])

    return gather_kernel(table, idx)


def kernel(x, embed):
    xf = x.reshape(NT, DIM)
    ef = embed.reshape(K, DIM)
    x2 = jnp.sum(x * x, axis=-1).reshape(NT, 1)
    e2 = jnp.sum(embed * embed, axis=-1).reshape(1, K)
    idx = _argmin_indices(xf, ef, x2, e2)
    idx_flat = idx.reshape(NT)
    quant = _sc_gather(ef, idx_flat)
    return quant.reshape(x.shape), idx_flat.reshape(x.shape[:-1])
